# Initial kernel scaffold; baseline (speedup 1.0000x reference)
import jax
import jax.numpy as jnp
from jax.experimental import pallas as pl


def kernel(node_feats, edge_feats, senders, receivers, W_en, b_en, W_ee, b_ee, W_e0, b_e0, W_n0, b_n0, W_g0, b_g0, W_e1, b_e1, W_n1, b_n1, W_g1, b_g1, W_dec, b_dec):
    raise NotImplementedError("write your pallas kernel here")


# speedup vs baseline: 2.7559x; 2.7559x over previous
import functools

import jax
import jax.numpy as jnp
from jax import lax
from jax.experimental import pallas as pl
from jax.experimental.pallas import tpu as pltpu
from jax.experimental.pallas import tpu_sc as plsc

N = 10000
E = 320000
H = 128

CH = 128
NCHUNK = E // CH
NC, NS = 2, 16
NW = NC * NS
NPAD = 10240
ROWS_PER_SUB = NPAD // NS

_F32 = jnp.float32
_sds = jax.ShapeDtypeStruct


_MESH = plsc.VectorSubcoreMesh(core_axis_name="c", subcore_axis_name="s")


@functools.partial(
    pl.kernel,
    out_type=_sds((E, H), _F32),
    mesh=_MESH,
    scratch_types=[
        pltpu.VMEM((CH,), jnp.int32),
        pltpu.VMEM((CH,), jnp.int32),
        pltpu.VMEM((CH, H), _F32),
        pltpu.SemaphoreType.DMA,
        pltpu.SemaphoreType.DMA,
    ],
)
def _edge_messages(a_hbm, ps_hbm, pr_hbm, s_hbm, r_hbm, m_hbm,
                   sidx, ridx, abuf, sem1, sem2):
    w = lax.axis_index("s") * NC + lax.axis_index("c")

    def chunk(i, _):
        k = w + i * NW
        base = k * CH
        pltpu.sync_copy(s_hbm.at[pl.ds(base, CH)], sidx)
        pltpu.sync_copy(r_hbm.at[pl.ds(base, CH)], ridx)
        pltpu.sync_copy(a_hbm.at[pl.ds(base, CH)], abuf)
        pltpu.async_copy(ps_hbm.at[sidx], abuf, sem1, add=True).wait()
        pltpu.async_copy(pr_hbm.at[ridx], abuf, sem2, add=True).wait()

        def row(rr, _):
            for j in range(H // 16):
                sl = pl.ds(j * 16, 16)
                abuf[rr, sl] = jnp.maximum(abuf[rr, sl], 0.0)
            return 0

        lax.fori_loop(0, CH, row, 0)
        pltpu.sync_copy(abuf, m_hbm.at[pl.ds(base, CH)])
        return 0

    nk = NCHUNK // NW + jnp.where(w < NCHUNK % NW, 1, 0)
    lax.fori_loop(0, nk, chunk, 0)


@functools.partial(
    pl.kernel,
    out_type=_sds((2, NPAD, H), _F32),
    mesh=_MESH,
    scratch_types=[
        pltpu.VMEM((CH,), jnp.int32),
        pltpu.VMEM((CH, H), _F32),
        pltpu.VMEM_SHARED((NPAD, H), _F32),
        pltpu.SemaphoreType.DMA,
    ],
)
def _segment_sums(m_hbm, idx2_hbm, zeros_hbm, out_hbm, idxv, rows, accum, sem):
    c = lax.axis_index("c")
    sid = lax.axis_index("s")

    pltpu.sync_copy(zeros_hbm.at[pl.ds(sid * ROWS_PER_SUB, ROWS_PER_SUB)],
                    accum.at[pl.ds(sid * ROWS_PER_SUB, ROWS_PER_SUB)])
    plsc.subcore_barrier()

    def chunk(i, _):
        k = sid + i * NS
        base = k * CH
        pltpu.sync_copy(idx2_hbm.at[c, pl.ds(base, CH)], idxv)
        pltpu.sync_copy(m_hbm.at[pl.ds(base, CH)], rows)
        pltpu.sync_copy(rows, accum.at[idxv], add=True)
        return 0

    nk = NCHUNK // NS + jnp.where(sid < NCHUNK % NS, 1, 0)
    lax.fori_loop(0, nk, chunk, 0)
    plsc.subcore_barrier()

    pltpu.sync_copy(accum.at[pl.ds(sid * ROWS_PER_SUB, ROWS_PER_SUB)],
                    out_hbm.at[c, pl.ds(sid * ROWS_PER_SUB, ROWS_PER_SUB)])


def _dot(a, b):
    return jnp.dot(a, b, preferred_element_type=_F32)


_BN = 1000
_GN = N // _BN
_BE = 4000
_GE = E // _BE

_full = lambda shape: pl.BlockSpec(shape, lambda i: tuple(0 for _ in shape))


def _prep_body(nf, wen, ben, wes, wer, nemb, ps, pr):
    nb = _dot(nf[...], wen[...]) + ben[...]
    nemb[...] = nb
    ps[...] = _dot(nb, wes[...])
    pr[...] = _dot(nb, wer[...])


def _k_prep(nf, wen, ben, wes, wer):
    return pl.pallas_call(
        _prep_body,
        grid=(_GN,),
        in_specs=[pl.BlockSpec((_BN, H), lambda i: (i, 0)),
                  _full((H, H)), _full((1, H)), _full((H, H)), _full((H, H))],
        out_specs=[pl.BlockSpec((_BN, H), lambda i: (i, 0))] * 3,
        out_shape=[_sds((N, H), _F32)] * 3,
    )(nf, wen, ben, wes, wer)


def _a0_body(ef, wee, bee, we0e, be0, a0):
    wc = _dot(wee[...], we0e[...])
    c0 = _dot(bee[...], we0e[...]) + be0[...]
    a0[...] = _dot(ef[...], wc) + c0


def _k_a0(ef, wee, bee, we0e, be0):
    return pl.pallas_call(
        _a0_body,
        grid=(_GE,),
        in_specs=[pl.BlockSpec((_BE, 16), lambda i: (i, 0)),
                  _full((16, H)), _full((1, H)), _full((H, H)), _full((1, H))],
        out_specs=pl.BlockSpec((_BE, H), lambda i: (i, 0)),
        out_shape=_sds((E, H), _F32),
    )(ef, wee, bee, we0e, be0)


def _a1_body(m0, glob1, we1e, we1g, be1, a1):
    row = _dot(glob1[...], we1g[...]) + be1[...]
    a1[...] = _dot(m0[...], we1e[...]) + row


def _k_a1(m0, glob1, we1e, we1g, be1):
    return pl.pallas_call(
        _a1_body,
        grid=(_GE,),
        in_specs=[pl.BlockSpec((_BE, H), lambda i: (i, 0)),
                  _full((1, H)), _full((H, H)), _full((H, H)), _full((1, H))],
        out_specs=pl.BlockSpec((_BE, H), lambda i: (i, 0)),
        out_shape=_sds((E, H), _F32),
    )(m0, glob1, we1e, we1g, be1)


def _node0_body(nemb, sr, wnn, wns, wnr, bn, wes, wer,
                nodes1, ps1, pr1, nagg, eagg):
    sent = sr[0]
    recv = sr[1]
    x = (_dot(nemb[...], wnn[...]) + _dot(sent, wns[...])
         + _dot(recv, wnr[...]) + bn[...])
    x = jnp.maximum(x, 0.0)
    nodes1[...] = x
    ps1[...] = _dot(x, wes[...])
    pr1[...] = _dot(x, wer[...])
    pn = jnp.sum(x, axis=0, keepdims=True)
    pe = jnp.sum(sent, axis=0, keepdims=True)

    @pl.when(pl.program_id(0) == 0)
    def _():
        nagg[...] = pn
        eagg[...] = pe

    @pl.when(pl.program_id(0) != 0)
    def _():
        nagg[...] += pn
        eagg[...] += pe


def _k_node0(nemb, sr, wnn, wns, wnr, bn, wes, wer):
    return pl.pallas_call(
        _node0_body,
        grid=(_GN,),
        in_specs=[pl.BlockSpec((_BN, H), lambda i: (i, 0)),
                  pl.BlockSpec((2, _BN, H), lambda i: (0, i, 0)),
                  _full((H, H)), _full((H, H)), _full((H, H)), _full((1, H)),
                  _full((H, H)), _full((H, H))],
        out_specs=[pl.BlockSpec((_BN, H), lambda i: (i, 0))] * 3
        + [pl.BlockSpec((1, H), lambda i: (0, 0))] * 2,
        out_shape=[_sds((N, H), _F32)] * 3 + [_sds((1, H), _F32)] * 2,
    )(nemb, sr, wnn, wns, wnr, bn, wes, wer)


def _glob1_body(nagg, eagg, wga, wgb, bg, glob1):
    glob1[...] = jnp.maximum(
        _dot(nagg[...], wga[...]) + _dot(eagg[...], wgb[...]) + bg[...], 0.0)


def _k_glob1(nagg, eagg, wga, wgb, bg):
    return pl.pallas_call(
        _glob1_body,
        out_shape=_sds((1, H), _F32),
    )(nagg, eagg, wga, wgb, bg)


def _node1_body(nodes1, sr, glob1, wnn, wns, wnr, wng, bn, nagg, eagg):
    sent = sr[0]
    recv = sr[1]
    grow = _dot(glob1[...], wng[...]) + bn[...]
    x = (_dot(nodes1[...], wnn[...]) + _dot(sent, wns[...])
         + _dot(recv, wnr[...]) + grow)
    x = jnp.maximum(x, 0.0)
    pn = jnp.sum(x, axis=0, keepdims=True)
    pe = jnp.sum(sent, axis=0, keepdims=True)

    @pl.when(pl.program_id(0) == 0)
    def _():
        nagg[...] = pn
        eagg[...] = pe

    @pl.when(pl.program_id(0) != 0)
    def _():
        nagg[...] += pn
        eagg[...] += pe


def _k_node1(nodes1, sr, glob1, wnn, wns, wnr, wng, bn):
    return pl.pallas_call(
        _node1_body,
        grid=(_GN,),
        in_specs=[pl.BlockSpec((_BN, H), lambda i: (i, 0)),
                  pl.BlockSpec((2, _BN, H), lambda i: (0, i, 0)),
                  _full((1, H)),
                  _full((H, H)), _full((H, H)), _full((H, H)), _full((H, H)),
                  _full((1, H))],
        out_specs=[pl.BlockSpec((1, H), lambda i: (0, 0))] * 2,
        out_shape=[_sds((1, H), _F32)] * 2,
    )(nodes1, sr, glob1, wnn, wns, wnr, wng, bn)


def _final_body(nagg, eagg, glob1, wga, wgb, wgc, bg, wdec_row, bdec, out):
    glob2 = jnp.maximum(
        _dot(nagg[...], wga[...]) + _dot(eagg[...], wgb[...])
        + _dot(glob1[...], wgc[...]) + bg[...], 0.0)
    out[...] = (jnp.sum(glob2 * wdec_row[...], axis=1, keepdims=True)
                + bdec[...])


def _k_final(nagg, eagg, glob1, wga, wgb, wgc, bg, wdec_row, bdec):
    return pl.pallas_call(
        _final_body,
        out_shape=_sds((1, 1), _F32),
    )(nagg, eagg, glob1, wga, wgb, wgc, bg, wdec_row, bdec)


def kernel(node_feats, edge_feats, senders, receivers, W_en, b_en, W_ee, b_ee,
           W_e0, b_e0, W_n0, b_n0, W_g0, b_g0,
           W_e1, b_e1, W_n1, b_n1, W_g1, b_g1,
           W_dec, b_dec):
    row = lambda b: b.reshape(1, -1)
    idx2 = jnp.stack([senders, receivers])
    zerosN = jnp.zeros((NPAD, H), _F32)

    we0e, we0s, we0r = W_e0[0:H], W_e0[H:2 * H], W_e0[2 * H:3 * H]
    wn0n, wn0s, wn0r = W_n0[0:H], W_n0[H:2 * H], W_n0[2 * H:3 * H]
    wg0a, wg0b = W_g0[0:H], W_g0[H:2 * H]
    we1e, we1s, we1r, we1g = W_e1[0:H], W_e1[H:2 * H], W_e1[2 * H:3 * H], W_e1[3 * H:4 * H]
    wn1n, wn1s, wn1r, wn1g = W_n1[0:H], W_n1[H:2 * H], W_n1[2 * H:3 * H], W_n1[3 * H:4 * H]
    wg1a, wg1b, wg1c = W_g1[0:H], W_g1[H:2 * H], W_g1[2 * H:3 * H]

    nemb, ps0, pr0 = _k_prep(node_feats, W_en, row(b_en), we0s, we0r)
    a0 = _k_a0(edge_feats, W_ee, row(b_ee), we0e, row(b_e0))
    m0 = _edge_messages(a0, ps0, pr0, senders, receivers)
    sr0 = _segment_sums(m0, idx2, zerosN)
    nodes1, ps1, pr1, nagg1, eagg1 = _k_node0(
        nemb, sr0, wn0n, wn0s, wn0r, row(b_n0), we1s, we1r)
    glob1 = _k_glob1(nagg1, eagg1, wg0a, wg0b, row(b_g0))
    a1 = _k_a1(m0, glob1, we1e, we1g, row(b_e1))
    m1 = _edge_messages(a1, ps1, pr1, senders, receivers)
    sr1 = _segment_sums(m1, idx2, zerosN)
    nagg2, eagg2 = _k_node1(
        nodes1, sr1, glob1, wn1n, wn1s, wn1r, wn1g, row(b_n1))
    return _k_final(nagg2, eagg2, glob1, wg1a, wg1b, wg1c, row(b_g1),
                    W_dec.reshape(1, H), row(b_dec))


# speedup vs baseline: 4.2435x; 1.5398x over previous
import functools

import jax
import jax.numpy as jnp
from jax import lax
from jax.experimental import pallas as pl
from jax.experimental.pallas import tpu as pltpu
from jax.experimental.pallas import tpu_sc as plsc

N = 10000
E = 320000
H = 128

CH = 128
NCHUNK = E // CH
NC, NS = 2, 16
NW = NC * NS
NPAD = 10240
ROWS_PER_SUB = NPAD // NS
EPS = E // NS
NKS = EPS // CH
REMS = EPS - NKS * CH

_F32 = jnp.float32
_sds = jax.ShapeDtypeStruct


_MESH = plsc.VectorSubcoreMesh(core_axis_name="c", subcore_axis_name="s")


EPW = E // NW
NKW = EPW // CH
REMW = EPW - NKW * CH


@functools.partial(
    pl.kernel,
    out_type=_sds((E, H), _F32),
    mesh=_MESH,
    scratch_types=[
        pltpu.VMEM((CH,), jnp.int32), pltpu.VMEM((CH,), jnp.int32),
        pltpu.VMEM((CH,), jnp.int32), pltpu.VMEM((CH,), jnp.int32),
        pltpu.VMEM((CH, H), _F32), pltpu.VMEM((CH, H), _F32),
        pltpu.SemaphoreType.DMA, pltpu.SemaphoreType.DMA,
        pltpu.SemaphoreType.DMA, pltpu.SemaphoreType.DMA,
        pltpu.SemaphoreType.DMA, pltpu.SemaphoreType.DMA,
    ],
)
def _edge_messages(a_hbm, ps_hbm, pr_hbm, s_hbm, r_hbm, m_hbm,
                   sidx0, ridx0, sidx1, ridx1, ab0, ab1,
                   semA0, semA1, semG0, semG1, semW0, semW1):
    w = lax.axis_index("s") * NC + lax.axis_index("c")
    base_w = w * EPW
    bufs = ((sidx0, ridx0, ab0, semA0, semG0, semW0),
            (sidx1, ridx1, ab1, semA1, semG1, semW1))

    def issue_in(k, B):
        sidx, ridx, ab, semA, _, _ = B
        base = base_w + k * CH
        pltpu.async_copy(s_hbm.at[pl.ds(base, CH)], sidx, semA)
        pltpu.async_copy(r_hbm.at[pl.ds(base, CH)], ridx, semA)
        pltpu.async_copy(a_hbm.at[pl.ds(base, CH)], ab, semA)

    def wait_in(k, B):
        sidx, ridx, ab, semA, _, _ = B
        base = base_w + k * CH
        pltpu.make_async_copy(s_hbm.at[pl.ds(base, CH)], sidx, semA).wait()
        pltpu.make_async_copy(r_hbm.at[pl.ds(base, CH)], ridx, semA).wait()
        pltpu.make_async_copy(a_hbm.at[pl.ds(base, CH)], ab, semA).wait()

    def relu(ab):
        def row(rr, _):
            for j in range(H // 16):
                sl = pl.ds(j * 16, 16)
                ab[rr, sl] = jnp.maximum(ab[rr, sl], 0.0)
            return 0
        lax.fori_loop(0, CH, row, 0)

    def process(k, B, issue_next):
        sidx, ridx, ab, semA, semG, semW = B
        base = base_w + k * CH
        wait_in(k, B)
        pltpu.async_copy(ps_hbm.at[sidx], ab, semG, add=True)
        pltpu.async_copy(pr_hbm.at[ridx], ab, semG, add=True)
        pltpu.make_async_copy(ps_hbm.at[sidx], ab, semG).wait()
        pltpu.make_async_copy(pr_hbm.at[ridx], ab, semG).wait()
        relu(ab)
        pltpu.async_copy(ab, m_hbm.at[pl.ds(base, CH)], semW)
        if issue_next:
            pltpu.make_async_copy(ab, m_hbm.at[pl.ds(base, CH)], semW).wait()
            issue_in(k + 2, B)

    issue_in(0, bufs[0])
    issue_in(1, bufs[1])

    def pair(p, _):
        for b, B in enumerate(bufs):
            process(2 * p + b, B, True)
        return 0

    lax.fori_loop(0, NKW // 2 - 1, pair, 0)
    for b, B in enumerate(bufs):
        k = NKW - 2 + b
        process(k, B, False)
        pltpu.make_async_copy(
            B[2], m_hbm.at[pl.ds(base_w + k * CH, CH)], B[5]).wait()

    base = base_w + NKW * CH
    sidx, ridx, ab, semA, semG, _ = bufs[0]
    pltpu.sync_copy(s_hbm.at[pl.ds(base, REMW)], sidx.at[pl.ds(0, REMW)])
    pltpu.sync_copy(r_hbm.at[pl.ds(base, REMW)], ridx.at[pl.ds(0, REMW)])
    pltpu.sync_copy(a_hbm.at[pl.ds(base, REMW)], ab.at[pl.ds(0, REMW)])
    pltpu.async_copy(ps_hbm.at[sidx.at[pl.ds(0, REMW)]],
                     ab.at[pl.ds(0, REMW)], semA, add=True).wait()
    pltpu.async_copy(pr_hbm.at[ridx.at[pl.ds(0, REMW)]],
                     ab.at[pl.ds(0, REMW)], semG, add=True).wait()

    def rrow(rr, _):
        for j in range(H // 16):
            sl = pl.ds(j * 16, 16)
            ab[rr, sl] = jnp.maximum(ab[rr, sl], 0.0)
        return 0

    lax.fori_loop(0, REMW, rrow, 0)
    pltpu.sync_copy(ab.at[pl.ds(0, REMW)], m_hbm.at[pl.ds(base, REMW)])


@functools.partial(
    pl.kernel,
    out_type=_sds((2, NPAD, H), _F32),
    mesh=_MESH,
    scratch_types=[
        pltpu.VMEM((CH,), jnp.int32), pltpu.VMEM((CH,), jnp.int32),
        pltpu.VMEM((CH, H), _F32), pltpu.VMEM((CH, H), _F32),
        pltpu.VMEM_SHARED((NPAD, H), _F32),
        pltpu.VMEM((REMS,), jnp.int32), pltpu.VMEM((REMS, H), _F32),
        pltpu.SemaphoreType.DMA, pltpu.SemaphoreType.DMA,
        pltpu.SemaphoreType.DMA, pltpu.SemaphoreType.DMA,
    ],
)
def _segment_sums(m_hbm, idx2_hbm, zeros_hbm, out_hbm,
                  idx0, idx1, rb0, rb1, accum, idxr, rowsr,
                  semI0, semI1, semS0, semS1):
    c = lax.axis_index("c")
    sid = lax.axis_index("s")

    pltpu.sync_copy(zeros_hbm.at[pl.ds(sid * ROWS_PER_SUB, ROWS_PER_SUB)],
                    accum.at[pl.ds(sid * ROWS_PER_SUB, ROWS_PER_SUB)])
    plsc.subcore_barrier()

    base_t = sid * EPS
    bufs = ((idx0, rb0, semI0, semS0), (idx1, rb1, semI1, semS1))

    def issue_in(k, B):
        idxv, rows, semI, _ = B
        base = base_t + k * CH
        pltpu.async_copy(idx2_hbm.at[pl.ds(c * E + base, CH)], idxv, semI)
        pltpu.async_copy(m_hbm.at[pl.ds(base, CH)], rows, semI)

    def process(k, B, issue_next):
        idxv, rows, semI, semS = B
        base = base_t + k * CH
        pltpu.make_async_copy(
            idx2_hbm.at[pl.ds(c * E + base, CH)], idxv, semI).wait()
        pltpu.make_async_copy(m_hbm.at[pl.ds(base, CH)], rows, semI).wait()
        pltpu.async_copy(rows, accum.at[idxv], semS, add=True)
        if issue_next:
            pltpu.make_async_copy(rows, accum.at[idxv], semS).wait()
            issue_in(k + 2, B)

    issue_in(0, bufs[0])
    issue_in(1, bufs[1])

    def pair(p, _):
        for b, B in enumerate(bufs):
            process(2 * p + b, B, True)
        return 0

    lax.fori_loop(0, NKS // 2 - 1, pair, 0)
    for b, B in enumerate(bufs):
        k = NKS - 2 + b
        process(k, B, False)
        pltpu.make_async_copy(B[1], accum.at[B[0]], B[3]).wait()

    base = base_t + NKS * CH
    semS = bufs[0][3]
    pltpu.sync_copy(idx2_hbm.at[pl.ds(c * E + base, REMS)], idxr)
    pltpu.sync_copy(m_hbm.at[pl.ds(base, REMS)], rowsr)
    pltpu.async_copy(rowsr, accum.at[idxr], semS, add=True).wait()

    plsc.subcore_barrier()
    pltpu.sync_copy(accum.at[pl.ds(sid * ROWS_PER_SUB, ROWS_PER_SUB)],
                    out_hbm.at[c, pl.ds(sid * ROWS_PER_SUB, ROWS_PER_SUB)])


def _dot(a, b):
    return jnp.dot(a, b, preferred_element_type=_F32)


_BN = 1000
_GN = N // _BN
_BE = 4000
_GE = E // _BE

_full = lambda shape: pl.BlockSpec(shape, lambda i: tuple(0 for _ in shape))


def _prep_body(nf, wen, ben, wes, wer, nemb, ps, pr):
    nb = _dot(nf[...], wen[...]) + ben[...]
    nemb[...] = nb
    ps[...] = _dot(nb, wes[...])
    pr[...] = _dot(nb, wer[...])


def _k_prep(nf, wen, ben, wes, wer):
    return pl.pallas_call(
        _prep_body,
        grid=(_GN,),
        in_specs=[pl.BlockSpec((_BN, H), lambda i: (i, 0)),
                  _full((H, H)), _full((1, H)), _full((H, H)), _full((H, H))],
        out_specs=[pl.BlockSpec((_BN, H), lambda i: (i, 0))] * 3,
        out_shape=[_sds((N, H), _F32)] * 3,
    )(nf, wen, ben, wes, wer)


def _a0_body(ef, wee, bee, we0e, be0, a0):
    wc = _dot(wee[...], we0e[...])
    c0 = _dot(bee[...], we0e[...]) + be0[...]
    a0[...] = _dot(ef[...], wc) + c0


def _k_a0(ef, wee, bee, we0e, be0):
    return pl.pallas_call(
        _a0_body,
        grid=(_GE,),
        in_specs=[pl.BlockSpec((_BE, 16), lambda i: (i, 0)),
                  _full((16, H)), _full((1, H)), _full((H, H)), _full((1, H))],
        out_specs=pl.BlockSpec((_BE, H), lambda i: (i, 0)),
        out_shape=_sds((E, H), _F32),
    )(ef, wee, bee, we0e, be0)


def _a1_body(m0, glob1, we1e, we1g, be1, a1):
    row = _dot(glob1[...], we1g[...]) + be1[...]
    a1[...] = _dot(m0[...], we1e[...]) + row


def _k_a1(m0, glob1, we1e, we1g, be1):
    return pl.pallas_call(
        _a1_body,
        grid=(_GE,),
        in_specs=[pl.BlockSpec((_BE, H), lambda i: (i, 0)),
                  _full((1, H)), _full((H, H)), _full((H, H)), _full((1, H))],
        out_specs=pl.BlockSpec((_BE, H), lambda i: (i, 0)),
        out_shape=_sds((E, H), _F32),
    )(m0, glob1, we1e, we1g, be1)


def _node0_body(nemb, sr, wnn, wns, wnr, bn, wes, wer,
                nodes1, ps1, pr1, nagg, eagg):
    sent = sr[0]
    recv = sr[1]
    x = (_dot(nemb[...], wnn[...]) + _dot(sent, wns[...])
         + _dot(recv, wnr[...]) + bn[...])
    x = jnp.maximum(x, 0.0)
    nodes1[...] = x
    ps1[...] = _dot(x, wes[...])
    pr1[...] = _dot(x, wer[...])
    pn = jnp.sum(x, axis=0, keepdims=True)
    pe = jnp.sum(sent, axis=0, keepdims=True)

    @pl.when(pl.program_id(0) == 0)
    def _():
        nagg[...] = pn
        eagg[...] = pe

    @pl.when(pl.program_id(0) != 0)
    def _():
        nagg[...] += pn
        eagg[...] += pe


def _k_node0(nemb, sr, wnn, wns, wnr, bn, wes, wer):
    return pl.pallas_call(
        _node0_body,
        grid=(_GN,),
        in_specs=[pl.BlockSpec((_BN, H), lambda i: (i, 0)),
                  pl.BlockSpec((2, _BN, H), lambda i: (0, i, 0)),
                  _full((H, H)), _full((H, H)), _full((H, H)), _full((1, H)),
                  _full((H, H)), _full((H, H))],
        out_specs=[pl.BlockSpec((_BN, H), lambda i: (i, 0))] * 3
        + [pl.BlockSpec((1, H), lambda i: (0, 0))] * 2,
        out_shape=[_sds((N, H), _F32)] * 3 + [_sds((1, H), _F32)] * 2,
    )(nemb, sr, wnn, wns, wnr, bn, wes, wer)


def _glob1_body(nagg, eagg, wga, wgb, bg, glob1):
    glob1[...] = jnp.maximum(
        _dot(nagg[...], wga[...]) + _dot(eagg[...], wgb[...]) + bg[...], 0.0)


def _k_glob1(nagg, eagg, wga, wgb, bg):
    return pl.pallas_call(
        _glob1_body,
        out_shape=_sds((1, H), _F32),
    )(nagg, eagg, wga, wgb, bg)


def _node1_body(nodes1, sr, glob1, wnn, wns, wnr, wng, bn, nagg, eagg):
    sent = sr[0]
    recv = sr[1]
    grow = _dot(glob1[...], wng[...]) + bn[...]
    x = (_dot(nodes1[...], wnn[...]) + _dot(sent, wns[...])
         + _dot(recv, wnr[...]) + grow)
    x = jnp.maximum(x, 0.0)
    pn = jnp.sum(x, axis=0, keepdims=True)
    pe = jnp.sum(sent, axis=0, keepdims=True)

    @pl.when(pl.program_id(0) == 0)
    def _():
        nagg[...] = pn
        eagg[...] = pe

    @pl.when(pl.program_id(0) != 0)
    def _():
        nagg[...] += pn
        eagg[...] += pe


def _k_node1(nodes1, sr, glob1, wnn, wns, wnr, wng, bn):
    return pl.pallas_call(
        _node1_body,
        grid=(_GN,),
        in_specs=[pl.BlockSpec((_BN, H), lambda i: (i, 0)),
                  pl.BlockSpec((2, _BN, H), lambda i: (0, i, 0)),
                  _full((1, H)),
                  _full((H, H)), _full((H, H)), _full((H, H)), _full((H, H)),
                  _full((1, H))],
        out_specs=[pl.BlockSpec((1, H), lambda i: (0, 0))] * 2,
        out_shape=[_sds((1, H), _F32)] * 2,
    )(nodes1, sr, glob1, wnn, wns, wnr, wng, bn)


def _final_body(nagg, eagg, glob1, wga, wgb, wgc, bg, wdec_row, bdec, out):
    glob2 = jnp.maximum(
        _dot(nagg[...], wga[...]) + _dot(eagg[...], wgb[...])
        + _dot(glob1[...], wgc[...]) + bg[...], 0.0)
    out[...] = (jnp.sum(glob2 * wdec_row[...], axis=1, keepdims=True)
                + bdec[...])


def _k_final(nagg, eagg, glob1, wga, wgb, wgc, bg, wdec_row, bdec):
    return pl.pallas_call(
        _final_body,
        out_shape=_sds((1, 1), _F32),
    )(nagg, eagg, glob1, wga, wgb, wgc, bg, wdec_row, bdec)


def kernel(node_feats, edge_feats, senders, receivers, W_en, b_en, W_ee, b_ee,
           W_e0, b_e0, W_n0, b_n0, W_g0, b_g0,
           W_e1, b_e1, W_n1, b_n1, W_g1, b_g1,
           W_dec, b_dec):
    row = lambda b: b.reshape(1, -1)
    idx2 = jnp.concatenate([senders, receivers])
    zerosN = jnp.zeros((NPAD, H), _F32)

    we0e, we0s, we0r = W_e0[0:H], W_e0[H:2 * H], W_e0[2 * H:3 * H]
    wn0n, wn0s, wn0r = W_n0[0:H], W_n0[H:2 * H], W_n0[2 * H:3 * H]
    wg0a, wg0b = W_g0[0:H], W_g0[H:2 * H]
    we1e, we1s, we1r, we1g = W_e1[0:H], W_e1[H:2 * H], W_e1[2 * H:3 * H], W_e1[3 * H:4 * H]
    wn1n, wn1s, wn1r, wn1g = W_n1[0:H], W_n1[H:2 * H], W_n1[2 * H:3 * H], W_n1[3 * H:4 * H]
    wg1a, wg1b, wg1c = W_g1[0:H], W_g1[H:2 * H], W_g1[2 * H:3 * H]

    nemb, ps0, pr0 = _k_prep(node_feats, W_en, row(b_en), we0s, we0r)
    a0 = _k_a0(edge_feats, W_ee, row(b_ee), we0e, row(b_e0))
    m0 = _edge_messages(a0, ps0, pr0, senders, receivers)
    sr0 = _segment_sums(m0, idx2, zerosN)
    nodes1, ps1, pr1, nagg1, eagg1 = _k_node0(
        nemb, sr0, wn0n, wn0s, wn0r, row(b_n0), we1s, we1r)
    glob1 = _k_glob1(nagg1, eagg1, wg0a, wg0b, row(b_g0))
    a1 = _k_a1(m0, glob1, we1e, we1g, row(b_e1))
    m1 = _edge_messages(a1, ps1, pr1, senders, receivers)
    sr1 = _segment_sums(m1, idx2, zerosN)
    nagg2, eagg2 = _k_node1(
        nodes1, sr1, glob1, wn1n, wn1s, wn1r, wn1g, row(b_n1))
    return _k_final(nagg2, eagg2, glob1, wg1a, wg1b, wg1c, row(b_g1),
                    W_dec.reshape(1, H), row(b_dec))


# speedup vs baseline: 4.3766x; 1.0314x over previous
import functools

import jax
import jax.numpy as jnp
from jax import lax
from jax.experimental import pallas as pl
from jax.experimental.pallas import tpu as pltpu
from jax.experimental.pallas import tpu_sc as plsc

N = 10000
E = 320000
H = 128

CH = 128
NC, NS = 2, 16
NW = NC * NS
NPAD = 10240
ROWS_PER_SUB = NPAD // NS
EPS = E // NS
NKS = EPS // CH
REMS = EPS - NKS * CH
EPW = E // NW
NKW = EPW // CH
REMW = EPW - NKW * CH

_F32 = jnp.float32
_sds = jax.ShapeDtypeStruct


_MESH = plsc.VectorSubcoreMesh(core_axis_name="c", subcore_axis_name="s")


@functools.partial(
    pl.kernel,
    out_type=(_sds((E, H), _F32), _sds((2, NPAD, H), _F32)),
    mesh=_MESH,
    scratch_types=[
        pltpu.VMEM((CH,), jnp.int32), pltpu.VMEM((CH,), jnp.int32),
        pltpu.VMEM((CH,), jnp.int32), pltpu.VMEM((CH,), jnp.int32),
        pltpu.VMEM((CH,), jnp.int32), pltpu.VMEM((CH,), jnp.int32),
        pltpu.VMEM((CH, H), _F32), pltpu.VMEM((CH, H), _F32),
        pltpu.VMEM_SHARED((NPAD, H), _F32),
        pltpu.VMEM((REMW,), jnp.int32),
        pltpu.SemaphoreType.DMA, pltpu.SemaphoreType.DMA,
        pltpu.SemaphoreType.DMA, pltpu.SemaphoreType.DMA,
        pltpu.SemaphoreType.DMA, pltpu.SemaphoreType.DMA,
        pltpu.SemaphoreType.DMA, pltpu.SemaphoreType.DMA,
    ],
)
def _edge_messages(a_hbm, ps_hbm, pr_hbm, s_hbm, r_hbm, idx2_hbm, zeros_hbm,
                   m_hbm, part_hbm,
                   sidx0, ridx0, sidx1, ridx1, kidx0, kidx1, ab0, ab1,
                   accum, kidxr,
                   semA0, semA1, semG0, semG1, semW0, semW1, semS0, semS1):
    c = lax.axis_index("c")
    sid = lax.axis_index("s")
    w = sid * NC + c
    base_w = w * EPW
    bufs = ((sidx0, ridx0, kidx0, ab0, semA0, semG0, semW0, semS0),
            (sidx1, ridx1, kidx1, ab1, semA1, semG1, semW1, semS1))

    pltpu.sync_copy(zeros_hbm.at[pl.ds(sid * ROWS_PER_SUB, ROWS_PER_SUB)],
                    accum.at[pl.ds(sid * ROWS_PER_SUB, ROWS_PER_SUB)])
    plsc.subcore_barrier()

    def issue_in(k, B):
        sidx, ridx, kidx, ab, semA = B[0], B[1], B[2], B[3], B[4]
        base = base_w + k * CH
        pltpu.async_copy(s_hbm.at[pl.ds(base, CH)], sidx, semA)
        pltpu.async_copy(r_hbm.at[pl.ds(base, CH)], ridx, semA)
        pltpu.async_copy(idx2_hbm.at[pl.ds(c * E + base, CH)], kidx, semA)
        pltpu.async_copy(a_hbm.at[pl.ds(base, CH)], ab, semA)

    def wait_in(k, B):
        sidx, ridx, kidx, ab, semA = B[0], B[1], B[2], B[3], B[4]
        base = base_w + k * CH
        pltpu.make_async_copy(s_hbm.at[pl.ds(base, CH)], sidx, semA).wait()
        pltpu.make_async_copy(r_hbm.at[pl.ds(base, CH)], ridx, semA).wait()
        pltpu.make_async_copy(
            idx2_hbm.at[pl.ds(c * E + base, CH)], kidx, semA).wait()
        pltpu.make_async_copy(a_hbm.at[pl.ds(base, CH)], ab, semA).wait()

    def relu(ab):
        def row(rr, _):
            for j in range(H // 16):
                sl = pl.ds(j * 16, 16)
                ab[rr, sl] = jnp.maximum(ab[rr, sl], 0.0)
            return 0
        lax.fori_loop(0, CH, row, 0)

    def process(k, B, issue_next):
        sidx, ridx, kidx, ab, semA, semG, semW, semS = B
        base = base_w + k * CH
        wait_in(k, B)
        pltpu.async_copy(ps_hbm.at[sidx], ab, semG, add=True)
        pltpu.async_copy(pr_hbm.at[ridx], ab, semG, add=True)
        pltpu.make_async_copy(ps_hbm.at[sidx], ab, semG).wait()
        pltpu.make_async_copy(pr_hbm.at[ridx], ab, semG).wait()
        relu(ab)
        pltpu.async_copy(ab, m_hbm.at[pl.ds(base, CH)], semW)
        pltpu.async_copy(ab, accum.at[kidx], semS, add=True)
        if issue_next:
            pltpu.make_async_copy(ab, m_hbm.at[pl.ds(base, CH)], semW).wait()
            pltpu.make_async_copy(ab, accum.at[kidx], semS).wait()
            issue_in(k + 2, B)

    issue_in(0, bufs[0])
    issue_in(1, bufs[1])

    def pair(p, _):
        for b, B in enumerate(bufs):
            process(2 * p + b, B, True)
        return 0

    lax.fori_loop(0, NKW // 2 - 1, pair, 0)
    for b, B in enumerate(bufs):
        k = NKW - 2 + b
        process(k, B, False)
        pltpu.make_async_copy(
            B[3], m_hbm.at[pl.ds(base_w + k * CH, CH)], B[6]).wait()
        pltpu.make_async_copy(B[3], accum.at[B[2]], B[7]).wait()

    base = base_w + NKW * CH
    sidx, ridx, kidx, ab, semA, semG, semW, semS = bufs[0]
    pltpu.sync_copy(s_hbm.at[pl.ds(base, REMW)], sidx.at[pl.ds(0, REMW)])
    pltpu.sync_copy(r_hbm.at[pl.ds(base, REMW)], ridx.at[pl.ds(0, REMW)])
    pltpu.sync_copy(idx2_hbm.at[pl.ds(c * E + base, REMW)], kidxr)
    pltpu.sync_copy(a_hbm.at[pl.ds(base, REMW)], ab.at[pl.ds(0, REMW)])
    pltpu.async_copy(ps_hbm.at[sidx.at[pl.ds(0, REMW)]],
                     ab.at[pl.ds(0, REMW)], semA, add=True).wait()
    pltpu.async_copy(pr_hbm.at[ridx.at[pl.ds(0, REMW)]],
                     ab.at[pl.ds(0, REMW)], semG, add=True).wait()

    def rrow(rr, _):
        for j in range(H // 16):
            sl = pl.ds(j * 16, 16)
            ab[rr, sl] = jnp.maximum(ab[rr, sl], 0.0)
        return 0

    lax.fori_loop(0, REMW, rrow, 0)
    pltpu.sync_copy(ab.at[pl.ds(0, REMW)], m_hbm.at[pl.ds(base, REMW)])
    pltpu.async_copy(ab.at[pl.ds(0, REMW)], accum.at[kidxr], semS,
                     add=True).wait()

    plsc.subcore_barrier()
    pltpu.sync_copy(accum.at[pl.ds(sid * ROWS_PER_SUB, ROWS_PER_SUB)],
                    part_hbm.at[c, pl.ds(sid * ROWS_PER_SUB, ROWS_PER_SUB)])


@functools.partial(
    pl.kernel,
    out_type=_sds((2, NPAD, H), _F32),
    mesh=_MESH,
    scratch_types=[
        pltpu.VMEM((CH,), jnp.int32), pltpu.VMEM((CH,), jnp.int32),
        pltpu.VMEM((CH, H), _F32), pltpu.VMEM((CH, H), _F32),
        pltpu.VMEM_SHARED((NPAD, H), _F32),
        pltpu.VMEM((REMW,), jnp.int32), pltpu.VMEM((REMW, H), _F32),
        pltpu.SemaphoreType.DMA, pltpu.SemaphoreType.DMA,
        pltpu.SemaphoreType.DMA, pltpu.SemaphoreType.DMA,
    ],
)
def _segment_sums(m_hbm, idx2_hbm, part_hbm, out_hbm,
                  idx0, idx1, rb0, rb1, accum, idxr, rowsr,
                  semI0, semI1, semS0, semS1):
    c = lax.axis_index("c")
    sid = lax.axis_index("s")

    pltpu.sync_copy(part_hbm.at[c, pl.ds(sid * ROWS_PER_SUB, ROWS_PER_SUB)],
                    accum.at[pl.ds(sid * ROWS_PER_SUB, ROWS_PER_SUB)])
    plsc.subcore_barrier()

    base_t = (sid * NC + (1 - c)) * EPW
    bufs = ((idx0, rb0, semI0, semS0), (idx1, rb1, semI1, semS1))

    def issue_in(k, B):
        idxv, rows, semI, _ = B
        base = base_t + k * CH
        pltpu.async_copy(idx2_hbm.at[pl.ds(c * E + base, CH)], idxv, semI)
        pltpu.async_copy(m_hbm.at[pl.ds(base, CH)], rows, semI)

    def process(k, B, issue_next):
        idxv, rows, semI, semS = B
        base = base_t + k * CH
        pltpu.make_async_copy(
            idx2_hbm.at[pl.ds(c * E + base, CH)], idxv, semI).wait()
        pltpu.make_async_copy(m_hbm.at[pl.ds(base, CH)], rows, semI).wait()
        pltpu.async_copy(rows, accum.at[idxv], semS, add=True)
        if issue_next:
            pltpu.make_async_copy(rows, accum.at[idxv], semS).wait()
            issue_in(k + 2, B)

    issue_in(0, bufs[0])
    issue_in(1, bufs[1])

    def pair(p, _):
        for b, B in enumerate(bufs):
            process(2 * p + b, B, True)
        return 0

    lax.fori_loop(0, NKW // 2 - 1, pair, 0)
    for b, B in enumerate(bufs):
        k = NKW - 2 + b
        process(k, B, False)
        pltpu.make_async_copy(B[1], accum.at[B[0]], B[3]).wait()

    base = base_t + NKW * CH
    semS = bufs[0][3]
    pltpu.sync_copy(idx2_hbm.at[pl.ds(c * E + base, REMW)], idxr)
    pltpu.sync_copy(m_hbm.at[pl.ds(base, REMW)], rowsr)
    pltpu.async_copy(rowsr, accum.at[idxr], semS, add=True).wait()

    plsc.subcore_barrier()
    pltpu.sync_copy(accum.at[pl.ds(sid * ROWS_PER_SUB, ROWS_PER_SUB)],
                    out_hbm.at[c, pl.ds(sid * ROWS_PER_SUB, ROWS_PER_SUB)])


def _dot(a, b):
    return jnp.dot(a.astype(jnp.bfloat16), b.astype(jnp.bfloat16),
                   preferred_element_type=_F32)


_BN = 1000
_GN = N // _BN
_BE = 4000
_GE = E // _BE

_full = lambda shape: pl.BlockSpec(shape, lambda i: tuple(0 for _ in shape))


def _prep_body(nf, wen, ben, wes, wer, nemb, ps, pr):
    nb = _dot(nf[...], wen[...]) + ben[...]
    nemb[...] = nb
    ps[...] = _dot(nb, wes[...])
    pr[...] = _dot(nb, wer[...])


def _k_prep(nf, wen, ben, wes, wer):
    return pl.pallas_call(
        _prep_body,
        grid=(_GN,),
        in_specs=[pl.BlockSpec((_BN, H), lambda i: (i, 0)),
                  _full((H, H)), _full((1, H)), _full((H, H)), _full((H, H))],
        out_specs=[pl.BlockSpec((_BN, H), lambda i: (i, 0))] * 3,
        out_shape=[_sds((N, H), _F32)] * 3,
    )(nf, wen, ben, wes, wer)


def _a0_body(ef, wee, bee, we0e, be0, a0):
    emb = _dot(ef[...], wee[...]) + bee[...]
    a0[...] = _dot(emb, we0e[...]) + be0[...]


def _k_a0(ef, wee, bee, we0e, be0):
    return pl.pallas_call(
        _a0_body,
        grid=(_GE,),
        in_specs=[pl.BlockSpec((_BE, H), lambda i: (i, 0)),
                  _full((H, H)), _full((1, H)), _full((H, H)), _full((1, H))],
        out_specs=pl.BlockSpec((_BE, H), lambda i: (i, 0)),
        out_shape=_sds((E, H), _F32),
    )(ef, wee, bee, we0e, be0)


def _a1_body(m0, glob1, we1e, we1g, be1, a1):
    a1[...] = (_dot(m0[...], we1e[...]) + _dot(glob1[...], we1g[...])
               + be1[...])


def _k_a1(m0, glob1, we1e, we1g, be1):
    return pl.pallas_call(
        _a1_body,
        grid=(_GE,),
        in_specs=[pl.BlockSpec((_BE, H), lambda i: (i, 0)),
                  _full((1, H)), _full((H, H)), _full((H, H)), _full((1, H))],
        out_specs=pl.BlockSpec((_BE, H), lambda i: (i, 0)),
        out_shape=_sds((E, H), _F32),
    )(m0, glob1, we1e, we1g, be1)


def _node0_body(nemb, sr, wnn, wns, wnr, bn, wes, wer,
                nodes1, ps1, pr1, nagg, eagg):
    sent = sr[0]
    recv = sr[1]
    x = (_dot(nemb[...], wnn[...]) + _dot(sent, wns[...])
         + _dot(recv, wnr[...]) + bn[...])
    x = jnp.maximum(x, 0.0)
    nodes1[...] = x
    ps1[...] = _dot(x, wes[...])
    pr1[...] = _dot(x, wer[...])
    pn = jnp.sum(x, axis=0, keepdims=True)
    pe = jnp.sum(sent, axis=0, keepdims=True)

    @pl.when(pl.program_id(0) == 0)
    def _():
        nagg[...] = pn
        eagg[...] = pe

    @pl.when(pl.program_id(0) != 0)
    def _():
        nagg[...] += pn
        eagg[...] += pe


def _k_node0(nemb, sr, wnn, wns, wnr, bn, wes, wer):
    return pl.pallas_call(
        _node0_body,
        grid=(_GN,),
        in_specs=[pl.BlockSpec((_BN, H), lambda i: (i, 0)),
                  pl.BlockSpec((2, _BN, H), lambda i: (0, i, 0)),
                  _full((H, H)), _full((H, H)), _full((H, H)), _full((1, H)),
                  _full((H, H)), _full((H, H))],
        out_specs=[pl.BlockSpec((_BN, H), lambda i: (i, 0))] * 3
        + [pl.BlockSpec((1, H), lambda i: (0, 0))] * 2,
        out_shape=[_sds((N, H), _F32)] * 3 + [_sds((1, H), _F32)] * 2,
    )(nemb, sr, wnn, wns, wnr, bn, wes, wer)


def _glob1_body(nagg, eagg, wga, wgb, bg, glob1):
    glob1[...] = jnp.maximum(
        _dot(nagg[...], wga[...]) + _dot(eagg[...], wgb[...]) + bg[...], 0.0)


def _k_glob1(nagg, eagg, wga, wgb, bg):
    return pl.pallas_call(
        _glob1_body,
        out_shape=_sds((1, H), _F32),
    )(nagg, eagg, wga, wgb, bg)


def _node1_body(nodes1, sr, glob1, wnn, wns, wnr, wng, bn, nagg, eagg):
    sent = sr[0]
    recv = sr[1]
    grow = _dot(glob1[...], wng[...]) + bn[...]
    x = (_dot(nodes1[...], wnn[...]) + _dot(sent, wns[...])
         + _dot(recv, wnr[...]) + grow)
    x = jnp.maximum(x, 0.0)
    pn = jnp.sum(x, axis=0, keepdims=True)
    pe = jnp.sum(sent, axis=0, keepdims=True)

    @pl.when(pl.program_id(0) == 0)
    def _():
        nagg[...] = pn
        eagg[...] = pe

    @pl.when(pl.program_id(0) != 0)
    def _():
        nagg[...] += pn
        eagg[...] += pe


def _k_node1(nodes1, sr, glob1, wnn, wns, wnr, wng, bn):
    return pl.pallas_call(
        _node1_body,
        grid=(_GN,),
        in_specs=[pl.BlockSpec((_BN, H), lambda i: (i, 0)),
                  pl.BlockSpec((2, _BN, H), lambda i: (0, i, 0)),
                  _full((1, H)),
                  _full((H, H)), _full((H, H)), _full((H, H)), _full((H, H)),
                  _full((1, H))],
        out_specs=[pl.BlockSpec((1, H), lambda i: (0, 0))] * 2,
        out_shape=[_sds((1, H), _F32)] * 2,
    )(nodes1, sr, glob1, wnn, wns, wnr, wng, bn)


def _final_body(nagg, eagg, glob1, wga, wgb, wgc, bg, wdec_row, bdec, out):
    glob2 = jnp.maximum(
        _dot(nagg[...], wga[...]) + _dot(eagg[...], wgb[...])
        + _dot(glob1[...], wgc[...]) + bg[...], 0.0)
    out[...] = (jnp.sum(glob2 * wdec_row[...], axis=1, keepdims=True)
                + bdec[...])


def _k_final(nagg, eagg, glob1, wga, wgb, wgc, bg, wdec_row, bdec):
    return pl.pallas_call(
        _final_body,
        out_shape=_sds((1, 1), _F32),
    )(nagg, eagg, glob1, wga, wgb, wgc, bg, wdec_row, bdec)


def kernel(node_feats, edge_feats, senders, receivers, W_en, b_en, W_ee, b_ee,
           W_e0, b_e0, W_n0, b_n0, W_g0, b_g0,
           W_e1, b_e1, W_n1, b_n1, W_g1, b_g1,
           W_dec, b_dec):
    row = lambda b: b.reshape(1, -1)
    idx2 = jnp.concatenate([senders, receivers])
    zerosN = jnp.zeros((NPAD, H), _F32)

    we0e, we0s, we0r = W_e0[0:H], W_e0[H:2 * H], W_e0[2 * H:3 * H]
    wn0n, wn0s, wn0r = W_n0[0:H], W_n0[H:2 * H], W_n0[2 * H:3 * H]
    wg0a, wg0b = W_g0[0:H], W_g0[H:2 * H]
    we1e, we1s, we1r, we1g = W_e1[0:H], W_e1[H:2 * H], W_e1[2 * H:3 * H], W_e1[3 * H:4 * H]
    wn1n, wn1s, wn1r, wn1g = W_n1[0:H], W_n1[H:2 * H], W_n1[2 * H:3 * H], W_n1[3 * H:4 * H]
    wg1a, wg1b, wg1c = W_g1[0:H], W_g1[H:2 * H], W_g1[2 * H:3 * H]

    nemb, ps0, pr0 = _k_prep(node_feats, W_en, row(b_en), we0s, we0r)
    ef_pad = jnp.pad(edge_feats, ((0, 0), (0, H - 16)))
    wee_pad = jnp.pad(W_ee, ((0, H - 16), (0, 0)))
    a0 = _k_a0(ef_pad, wee_pad, row(b_ee), we0e, row(b_e0))
    m0, part0 = _edge_messages(a0, ps0, pr0, senders, receivers, idx2, zerosN)
    sr0 = _segment_sums(m0, idx2, part0)
    nodes1, ps1, pr1, nagg1, eagg1 = _k_node0(
        nemb, sr0, wn0n, wn0s, wn0r, row(b_n0), we1s, we1r)
    glob1 = _k_glob1(nagg1, eagg1, wg0a, wg0b, row(b_g0))
    a1 = _k_a1(m0, glob1, we1e, we1g, row(b_e1))
    m1, part1 = _edge_messages(a1, ps1, pr1, senders, receivers, idx2, zerosN)
    sr1 = _segment_sums(m1, idx2, part1)
    nagg2, eagg2 = _k_node1(
        nodes1, sr1, glob1, wn1n, wn1s, wn1r, wn1g, row(b_n1))
    return _k_final(nagg2, eagg2, glob1, wg1a, wg1b, wg1c, row(b_g1),
                    W_dec.reshape(1, H), row(b_dec))
